# Initial kernel scaffold; baseline (speedup 1.0000x reference)
import jax
import jax.numpy as jnp
from jax.experimental import pallas as pl


def kernel(features_all, type_mask, edge_index, mp_idx_user, mp_idx_item, tgt_user, tgt_item, W_fc0, b_fc0, W_fc1, b_fc1, W_hg, attn_u, attn_i, W_user, b_user, W_item, b_item, W_xij, b_xij, g_xij, bt_xij, W_l1, b_l1, g_l1, bt_l1, W_l2, b_l2):
    raise NotImplementedError("write your pallas kernel here")


# speedup vs baseline: 4.1875x; 4.1875x over previous
import functools

import jax
import jax.numpy as jnp
from jax import lax
from jax.experimental import pallas as pl
from jax.experimental.pallas import tpu as pltpu
from jax.experimental.pallas import tpu_sc as plsc

N = 10000
NPAD = 10240
E = 320000
D = 128
H = 8
B = 1024
S = 16
L = 3

NC = 2
NS = 16
NW = NC * NS
EPT = E // NW
EPT_PAD = 10240
KJ = EPT_PAD // 128
ECH = 8
STRIPE = NPAD // NS

_f32 = jnp.float32


def _elu(x):
    return jnp.where(x > 0, x, jnp.exp(jnp.minimum(x, 0.0)) - 1.0)


def _proj_body(x_ref, m_ref, w0_ref, b0_ref, w1_ref, b1_ref, o_ref):
    x = x_ref[...]
    h0 = jnp.dot(x, w0_ref[...], preferred_element_type=_f32) + b0_ref[...]
    h1 = jnp.dot(x, w1_ref[...], preferred_element_type=_f32) + b1_ref[...]
    o_ref[...] = jnp.where(m_ref[...] == 0.0, h0, h1)


def _project(xp, tm, W0, b0, W1, b1):
    nblk = NPAD // 512
    return pl.pallas_call(
        _proj_body,
        grid=(nblk,),
        in_specs=[
            pl.BlockSpec((512, D), lambda i: (i, 0)),
            pl.BlockSpec((512, 1), lambda i: (i, 0)),
            pl.BlockSpec((D, D), lambda i: (0, 0)),
            pl.BlockSpec((1, D), lambda i: (0, 0)),
            pl.BlockSpec((D, D), lambda i: (0, 0)),
            pl.BlockSpec((1, D), lambda i: (0, 0)),
        ],
        out_specs=pl.BlockSpec((512, D), lambda i: (i, 0)),
        out_shape=jax.ShapeDtypeStruct((NPAD, D), _f32),
    )(xp, tm, W0, b0, W1, b1)


@functools.cache
def _sc_mesh():
    return plsc.VectorSubcoreMesh(core_axis_name="c", subcore_axis_name="s",
                                  num_cores=NC, num_subcores=NS)


@functools.cache
def _build_edge_agg():
    return functools.partial(
        pl.kernel,
        mesh=_sc_mesh(),
        out_type=[
            jax.ShapeDtypeStruct((NC, NPAD, D), _f32),
            jax.ShapeDtypeStruct((NC, NPAD), _f32),
        ],
        scratch_types=[
            pltpu.VMEM((ECH, 128), jnp.int32),
            pltpu.VMEM((ECH, 128), jnp.int32),
            pltpu.VMEM((ECH, 128), jnp.int32),
            pltpu.VMEM((128, D), _f32),
            pltpu.VMEM((128,), _f32),
            pltpu.VMEM_SHARED((NPAD, D), _f32),
            pltpu.VMEM_SHARED((NPAD,), _f32),
            pltpu.SemaphoreType.DMA,
        ],
    )(_edge_agg_body)


def _edge_agg_body(feats_hbm, ep_hbm, agg_out, deg_out,
                   ep_v, src_v, dst_v, rows, ones_v, agg_sh, deg_sh, sem):
    cid = lax.axis_index("c")
    sid = lax.axis_index("s")
    wid = sid * NC + cid

    def _zrow(r, carry):
        for c in range(D // 16):
            rows[r, pl.ds(c * 16, 16)] = jnp.zeros((16,), _f32)
        return carry

    lax.fori_loop(0, 128, _zrow, 0)
    for c in range(8):
        ones_v[pl.ds(c * 16, 16)] = jnp.ones((16,), _f32)

    r0 = sid * STRIPE
    for k in range(STRIPE // 128):
        pltpu.sync_copy(rows, agg_sh.at[pl.ds(r0 + k * 128, 128)])
        pltpu.sync_copy(rows.at[0], deg_sh.at[pl.ds(r0 + k * 128, 128)])
    plsc.subcore_barrier()

    def _chunk(cc, carry):
        pltpu.sync_copy(ep_hbm.at[wid, pl.ds(cc * ECH, ECH)], ep_v)
        for r in range(ECH):
            for c in range(128 // 16):
                sl = pl.ds(c * 16, 16)
                v = ep_v[r, sl]
                src_v[r, sl] = v & jnp.int32(16383)
                dst_v[r, sl] = lax.shift_right_logical(v, jnp.int32(14))
        for j in range(ECH):
            pltpu.async_copy(feats_hbm.at[src_v.at[j]], rows, sem).wait()
            pltpu.sync_copy(rows, agg_sh.at[dst_v.at[j]], add=True)
            pltpu.sync_copy(ones_v, deg_sh.at[dst_v.at[j]], add=True)
        return carry

    lax.fori_loop(0, KJ // ECH, _chunk, 0)
    plsc.subcore_barrier()

    pltpu.sync_copy(agg_sh.at[pl.ds(r0, STRIPE)],
                    agg_out.at[cid, pl.ds(r0, STRIPE)])
    pltpu.sync_copy(deg_sh.at[pl.ds(r0, STRIPE)],
                    deg_out.at[cid, pl.ds(r0, STRIPE)])


def _hgcn_body(f_ref, a_ref, d_ref, w_ref, o_ref):
    f = f_ref[...]
    a = a_ref[0] + a_ref[1]
    dg = jnp.maximum(d_ref[0] + d_ref[1], 1.0)
    y = jnp.dot(f + a / dg, w_ref[...], preferred_element_type=_f32)
    o_ref[...] = _elu(y)


def _hgcn(feats, aggp, degp, W_hg):
    nblk = NPAD // 512
    return pl.pallas_call(
        _hgcn_body,
        grid=(nblk,),
        in_specs=[
            pl.BlockSpec((512, D), lambda i: (i, 0)),
            pl.BlockSpec((NC, 512, D), lambda i: (0, i, 0)),
            pl.BlockSpec((NC, 512, 1), lambda i: (0, i, 0)),
            pl.BlockSpec((D, D), lambda i: (0, 0)),
        ],
        out_specs=pl.BlockSpec((512, D), lambda i: (i, 0)),
        out_shape=jax.ShapeDtypeStruct((NPAD, D), _f32),
    )(feats, aggp, degp, W_hg)


_ROWS_PER_W = (B * S) // NW
_JB = _ROWS_PER_W // 128
_TGT_PER_W = B // NW


@functools.cache
def _build_mp_gather():
    return functools.partial(
        pl.kernel,
        mesh=_sc_mesh(),
        out_type=[
            jax.ShapeDtypeStruct((B * S, D), _f32),
            jax.ShapeDtypeStruct((B * S, D), _f32),
            jax.ShapeDtypeStruct((B, D), _f32),
            jax.ShapeDtypeStruct((B, D), _f32),
        ],
        scratch_types=[
            pltpu.VMEM((L, _JB, 128), jnp.int32),
            pltpu.VMEM((128, D), _f32),
            pltpu.VMEM((128, D), _f32),
            pltpu.VMEM((128, D), _f32),
            pltpu.VMEM((128, D), _f32),
            pltpu.VMEM((_TGT_PER_W,), jnp.int32),
            pltpu.VMEM((_TGT_PER_W, D), _f32),
            pltpu.SemaphoreType.DMA,
        ],
    )(_mp_gather_body)


def _mp_gather_body(feats_hbm, mpu_hbm, mpi_hbm, tgtu_hbm, tgti_hbm,
                    embu_o, embi_o, tfu_o, tfi_o,
                    idxm, rb0, rb1, rb2, outb, tix, trows, sem):
    cid = lax.axis_index("c")
    sid = lax.axis_index("s")
    wid = sid * NC + cid

    for mp_h, emb_o in ((mpu_hbm, embu_o), (mpi_hbm, embi_o)):
        pltpu.sync_copy(mp_h.at[wid], idxm)
        for j in range(_JB):
            c0 = pltpu.async_copy(feats_hbm.at[idxm.at[0, j]], rb0, sem)
            c1 = pltpu.async_copy(feats_hbm.at[idxm.at[1, j]], rb1, sem)
            c2 = pltpu.async_copy(feats_hbm.at[idxm.at[2, j]], rb2, sem)
            c0.wait()
            c1.wait()
            c2.wait()

            def _srow(r, carry):
                for c in range(D // 16):
                    sl = pl.ds(c * 16, 16)
                    outb[r, sl] = (rb0[r, sl] + rb1[r, sl] + rb2[r, sl]) \
                        * _f32(1.0 / 3.0)
                return carry

            lax.fori_loop(0, 128, _srow, 0)
            pltpu.sync_copy(
                outb, emb_o.at[pl.ds(wid * _ROWS_PER_W + j * 128, 128)])

    for tg_h, tf_o in ((tgtu_hbm, tfu_o), (tgti_hbm, tfi_o)):
        pltpu.sync_copy(tg_h.at[wid], tix)
        pltpu.async_copy(feats_hbm.at[tix], trows, sem).wait()
        pltpu.sync_copy(trows, tf_o.at[pl.ds(wid * _TGT_PER_W, _TGT_PER_W)])


_BBLK = 256


def _ln_tc(x, g, b):
    m = jnp.mean(x, axis=-1, keepdims=True)
    v = jnp.mean((x - m) ** 2, axis=-1, keepdims=True)
    return (x - m) / jnp.sqrt(v + 1e-5) * g + b


def _attn_head(emb3, attn, tgtf, W, b):
    embf = emb3.reshape(_BBLK * S, D)
    e = jnp.dot(embf, attn, preferred_element_type=_f32)
    e = jnp.where(e >= 0, e, 0.01 * e)
    e3 = e.reshape(_BBLK, S, H)
    m = jnp.max(e3, axis=1, keepdims=True)
    ex = jnp.exp(e3 - m)
    a = ex / jnp.sum(ex, axis=1, keepdims=True)
    parts = []
    for h in range(H):
        w = a[:, :, h]
        parts.append(jnp.sum(w[:, :, None] * emb3, axis=1) + tgtf)
    x = _elu(jnp.concatenate(parts, axis=1))
    return jnp.dot(x, W, preferred_element_type=_f32) + b


def _tail_body(eu_ref, ei_ref, tu_ref, ti_ref, au_ref, ai_ref,
               wu_ref, bu_ref, wi_ref, bi_ref,
               wx_ref, bx_ref, gx_ref, btx_ref,
               w1_ref, b1_ref, g1_ref, bt1_ref,
               w2_ref, b2_ref, o_ref):
    hu = _attn_head(eu_ref[...], au_ref[...], tu_ref[...],
                    wu_ref[...], bu_ref[...])
    hi = _attn_head(ei_ref[...], ai_ref[...], ti_ref[...],
                    wi_ref[...], bi_ref[...])
    x = hu * hi
    x = _elu(_ln_tc(jnp.dot(x, wx_ref[...], preferred_element_type=_f32)
                    + bx_ref[...], gx_ref[...], btx_ref[...]))
    y = _elu(_ln_tc(jnp.dot(x, w1_ref[...], preferred_element_type=_f32)
                    + b1_ref[...], g1_ref[...], bt1_ref[...]))
    z = jnp.dot(y, w2_ref[...], preferred_element_type=_f32) + b2_ref[...]
    o_ref[...] = 1.0 / (1.0 + jnp.exp(-z))


def _tail(embu3, embi3, tfu, tfi, attn_u, attn_i, Wu, bu, Wi, bi,
          Wx, bx, gx, btx, W1, b1, g1, bt1, W2p, b2p):
    full = lambda shape: pl.BlockSpec(shape, lambda i: tuple(0 for _ in shape))
    return pl.pallas_call(
        _tail_body,
        grid=(B // _BBLK,),
        in_specs=[
            pl.BlockSpec((_BBLK, S, D), lambda i: (i, 0, 0)),
            pl.BlockSpec((_BBLK, S, D), lambda i: (i, 0, 0)),
            pl.BlockSpec((_BBLK, D), lambda i: (i, 0)),
            pl.BlockSpec((_BBLK, D), lambda i: (i, 0)),
            full((D, H)), full((D, H)),
            full((H * D, D)), full((1, D)),
            full((H * D, D)), full((1, D)),
            full((D, D)), full((1, D)), full((1, D)), full((1, D)),
            full((D, D)), full((1, D)), full((1, D)), full((1, D)),
            full((D, D)), full((1, D)),
        ],
        out_specs=pl.BlockSpec((_BBLK, D), lambda i: (i, 0)),
        out_shape=jax.ShapeDtypeStruct((B, D), _f32),
    )(embu3, embi3, tfu, tfi, attn_u, attn_i, Wu, bu, Wi, bi,
      Wx, bx, gx, btx, W1, b1, g1, bt1, W2p, b2p)


def kernel(features_all, type_mask, edge_index, mp_idx_user, mp_idx_item,
           tgt_user, tgt_item, W_fc0, b_fc0, W_fc1, b_fc1, W_hg, attn_u,
           attn_i, W_user, b_user, W_item, b_item, W_xij, b_xij, g_xij,
           bt_xij, W_l1, b_l1, g_l1, bt_l1, W_l2, b_l2):
    i32 = jnp.int32

    xp = jnp.pad(features_all, ((0, NPAD - N), (0, 0)))
    tm = jnp.pad(type_mask.astype(_f32), (0, NPAD - N)).reshape(NPAD, 1)

    src = edge_index[0].astype(i32).reshape(NW, EPT)
    dst = edge_index[1].astype(i32).reshape(NW, EPT)
    packed = src | (dst << 14)
    ep = jnp.concatenate(
        [packed, jnp.full((NW, EPT_PAD - EPT), N << 14, i32)],
        axis=1).reshape(NW, KJ, 128)

    mpu = mp_idx_user.astype(i32).reshape(NW, _JB, 128, L).transpose(0, 3, 1, 2)
    mpi = mp_idx_item.astype(i32).reshape(NW, _JB, 128, L).transpose(0, 3, 1, 2)
    tgtu2 = tgt_user.astype(i32).reshape(NW, _TGT_PER_W)
    tgti2 = tgt_item.astype(i32).reshape(NW, _TGT_PER_W)

    feats = _project(xp, tm, W_fc0, b_fc0.reshape(1, D),
                     W_fc1, b_fc1.reshape(1, D))
    aggp, degp = _build_edge_agg()(feats, ep)
    feats2 = _hgcn(feats, aggp, degp.reshape(NC, NPAD, 1), W_hg)

    embu, embi, tfu, tfi = _build_mp_gather()(feats2, mpu, mpi, tgtu2, tgti2)
    embu3 = embu.reshape(B, S, D)
    embi3 = embi.reshape(B, S, D)

    W2p = jnp.pad(W_l2, ((0, 0), (0, D - 1)))
    b2p = jnp.pad(b_l2, (0, D - 1)).reshape(1, D)

    out2d = _tail(embu3, embi3, tfu, tfi, attn_u, attn_i,
                  W_user, b_user.reshape(1, D), W_item, b_item.reshape(1, D),
                  W_xij, b_xij.reshape(1, D), g_xij.reshape(1, D),
                  bt_xij.reshape(1, D),
                  W_l1, b_l1.reshape(1, D), g_l1.reshape(1, D),
                  bt_l1.reshape(1, D), W2p, b2p)
    return out2d[:, 0]


# speedup vs baseline: 4.4883x; 1.0718x over previous
import functools

import jax
import jax.numpy as jnp
from jax import lax
from jax.experimental import pallas as pl
from jax.experimental.pallas import tpu as pltpu
from jax.experimental.pallas import tpu_sc as plsc

N = 10000
NPAD = 10240
E = 320000
D = 128
H = 8
B = 1024
S = 16
L = 3

NC = 2
NS = 16
NW = NC * NS
EPT = E // NW
EPT_PAD = 10240
KJ = EPT_PAD // 128
ECH = 8
STRIPE = NPAD // NS

_f32 = jnp.float32


def _elu(x):
    return jnp.where(x > 0, x, jnp.exp(jnp.minimum(x, 0.0)) - 1.0)


def _proj_body(x_ref, m_ref, w0_ref, b0_ref, w1_ref, b1_ref, o_ref):
    x = x_ref[...]
    h0 = jnp.dot(x, w0_ref[...], preferred_element_type=_f32) + b0_ref[...]
    h1 = jnp.dot(x, w1_ref[...], preferred_element_type=_f32) + b1_ref[...]
    o_ref[...] = jnp.where(m_ref[...] == 0.0, h0, h1)


def _project(xp, tm, W0, b0, W1, b1):
    nblk = NPAD // 512
    return pl.pallas_call(
        _proj_body,
        grid=(nblk,),
        in_specs=[
            pl.BlockSpec((512, D), lambda i: (i, 0)),
            pl.BlockSpec((512, 1), lambda i: (i, 0)),
            pl.BlockSpec((D, D), lambda i: (0, 0)),
            pl.BlockSpec((1, D), lambda i: (0, 0)),
            pl.BlockSpec((D, D), lambda i: (0, 0)),
            pl.BlockSpec((1, D), lambda i: (0, 0)),
        ],
        out_specs=pl.BlockSpec((512, D), lambda i: (i, 0)),
        out_shape=jax.ShapeDtypeStruct((NPAD, D), _f32),
    )(xp, tm, W0, b0, W1, b1)


@functools.cache
def _sc_mesh():
    return plsc.VectorSubcoreMesh(core_axis_name="c", subcore_axis_name="s",
                                  num_cores=NC, num_subcores=NS)


@functools.cache
def _build_edge_agg():
    return functools.partial(
        pl.kernel,
        mesh=_sc_mesh(),
        out_type=[
            jax.ShapeDtypeStruct((NC, NPAD, D), _f32),
            jax.ShapeDtypeStruct((NC, NPAD), _f32),
        ],
        scratch_types=[
            pltpu.VMEM((ECH, 128), jnp.int32),
            pltpu.VMEM((ECH, 128), jnp.int32),
            pltpu.VMEM((ECH, 128), jnp.int32),
            pltpu.VMEM((128, D), _f32),
            pltpu.VMEM((128, D), _f32),
            pltpu.VMEM((128,), _f32),
            pltpu.VMEM_SHARED((NPAD, D), _f32),
            pltpu.VMEM_SHARED((NPAD,), _f32),
            pltpu.SemaphoreType.DMA,
            pltpu.SemaphoreType.DMA,
            pltpu.SemaphoreType.DMA,
            pltpu.SemaphoreType.DMA,
        ],
    )(_edge_agg_body)


def _edge_agg_body(feats_hbm, ep_hbm, agg_out, deg_out,
                   ep_v, src_v, dst_v, rows0, rows1, ones_v, agg_sh, deg_sh,
                   sem_g, sem_s0, sem_s1, sem_d):
    cid = lax.axis_index("c")
    sid = lax.axis_index("s")
    wid = sid * NC + cid
    rbufs = (rows0, rows1)
    ssems = (sem_s0, sem_s1)

    def _zrow(r, carry):
        for c in range(D // 16):
            rows0[r, pl.ds(c * 16, 16)] = jnp.zeros((16,), _f32)
        return carry

    lax.fori_loop(0, 128, _zrow, 0)
    for c in range(8):
        ones_v[pl.ds(c * 16, 16)] = jnp.ones((16,), _f32)

    r0 = sid * STRIPE
    for k in range(STRIPE // 128):
        pltpu.sync_copy(rows0, agg_sh.at[pl.ds(r0 + k * 128, 128)])
        pltpu.sync_copy(rows0.at[0], deg_sh.at[pl.ds(r0 + k * 128, 128)])
    plsc.subcore_barrier()

    def _chunk(cc, carry):
        pltpu.sync_copy(ep_hbm.at[wid, pl.ds(cc * ECH, ECH)], ep_v)
        for r in range(ECH):
            for c in range(128 // 16):
                sl = pl.ds(c * 16, 16)
                v = ep_v[r, sl]
                src_v[r, sl] = v & jnp.int32(16383)
                dst_v[r, sl] = lax.shift_right_logical(v, jnp.int32(14))
        gd = pltpu.async_copy(feats_hbm.at[src_v.at[0]], rbufs[0], sem_g)
        sc_d = [None, None]
        deg_d = []
        for j in range(ECH):
            b = j % 2
            gd.wait()
            sc_d[b] = pltpu.async_copy(rbufs[b], agg_sh.at[dst_v.at[j]],
                                       ssems[b], add=True)
            deg_d.append(pltpu.async_copy(ones_v, deg_sh.at[dst_v.at[j]],
                                          sem_d, add=True))
            if j + 1 < ECH:
                nb = 1 - b
                if sc_d[nb] is not None:
                    sc_d[nb].wait()
                gd = pltpu.async_copy(feats_hbm.at[src_v.at[j + 1]],
                                     rbufs[nb], sem_g)
        sc_d[(ECH - 1) % 2].wait()
        if sc_d[ECH % 2] is not None:
            sc_d[ECH % 2].wait()
        for dd in deg_d:
            dd.wait()
        return carry

    lax.fori_loop(0, KJ // ECH, _chunk, 0)
    plsc.subcore_barrier()

    pltpu.sync_copy(agg_sh.at[pl.ds(r0, STRIPE)],
                    agg_out.at[cid, pl.ds(r0, STRIPE)])
    pltpu.sync_copy(deg_sh.at[pl.ds(r0, STRIPE)],
                    deg_out.at[cid, pl.ds(r0, STRIPE)])


def _hgcn_body(f_ref, a_ref, d_ref, w_ref, o_ref):
    f = f_ref[...]
    a = a_ref[0] + a_ref[1]
    dg = jnp.maximum(d_ref[0] + d_ref[1], 1.0)
    y = jnp.dot(f + a / dg, w_ref[...], preferred_element_type=_f32)
    o_ref[...] = _elu(y)


def _hgcn(feats, aggp, degp, W_hg):
    nblk = NPAD // 512
    return pl.pallas_call(
        _hgcn_body,
        grid=(nblk,),
        in_specs=[
            pl.BlockSpec((512, D), lambda i: (i, 0)),
            pl.BlockSpec((NC, 512, D), lambda i: (0, i, 0)),
            pl.BlockSpec((NC, 512, 1), lambda i: (0, i, 0)),
            pl.BlockSpec((D, D), lambda i: (0, 0)),
        ],
        out_specs=pl.BlockSpec((512, D), lambda i: (i, 0)),
        out_shape=jax.ShapeDtypeStruct((NPAD, D), _f32),
    )(feats, aggp, degp, W_hg)


_ROWS_PER_W = (B * S) // NW
_JB = _ROWS_PER_W // 128
_TGT_PER_W = B // NW


@functools.cache
def _build_mp_gather():
    return functools.partial(
        pl.kernel,
        mesh=_sc_mesh(),
        out_type=[
            jax.ShapeDtypeStruct((B * S, D), _f32),
            jax.ShapeDtypeStruct((B * S, D), _f32),
            jax.ShapeDtypeStruct((B, D), _f32),
            jax.ShapeDtypeStruct((B, D), _f32),
        ],
        scratch_types=[
            pltpu.VMEM((L, _JB, 128), jnp.int32),
            pltpu.VMEM((128, D), _f32),
            pltpu.VMEM((128, D), _f32),
            pltpu.VMEM((128, D), _f32),
            pltpu.VMEM((128, D), _f32),
            pltpu.VMEM((_TGT_PER_W,), jnp.int32),
            pltpu.VMEM((_TGT_PER_W, D), _f32),
            pltpu.SemaphoreType.DMA,
        ],
    )(_mp_gather_body)


def _mp_gather_body(feats_hbm, mpu_hbm, mpi_hbm, tgtu_hbm, tgti_hbm,
                    embu_o, embi_o, tfu_o, tfi_o,
                    idxm, rb0, rb1, rb2, outb, tix, trows, sem):
    cid = lax.axis_index("c")
    sid = lax.axis_index("s")
    wid = sid * NC + cid

    for mp_h, emb_o in ((mpu_hbm, embu_o), (mpi_hbm, embi_o)):
        pltpu.sync_copy(mp_h.at[wid], idxm)
        for j in range(_JB):
            c0 = pltpu.async_copy(feats_hbm.at[idxm.at[0, j]], rb0, sem)
            c1 = pltpu.async_copy(feats_hbm.at[idxm.at[1, j]], rb1, sem)
            c2 = pltpu.async_copy(feats_hbm.at[idxm.at[2, j]], rb2, sem)
            c0.wait()
            c1.wait()
            c2.wait()

            def _srow(r, carry):
                for c in range(D // 16):
                    sl = pl.ds(c * 16, 16)
                    outb[r, sl] = (rb0[r, sl] + rb1[r, sl] + rb2[r, sl]) \
                        * _f32(1.0 / 3.0)
                return carry

            lax.fori_loop(0, 128, _srow, 0)
            pltpu.sync_copy(
                outb, emb_o.at[pl.ds(wid * _ROWS_PER_W + j * 128, 128)])

    for tg_h, tf_o in ((tgtu_hbm, tfu_o), (tgti_hbm, tfi_o)):
        pltpu.sync_copy(tg_h.at[wid], tix)
        pltpu.async_copy(feats_hbm.at[tix], trows, sem).wait()
        pltpu.sync_copy(trows, tf_o.at[pl.ds(wid * _TGT_PER_W, _TGT_PER_W)])


_BBLK = 256


def _ln_tc(x, g, b):
    m = jnp.mean(x, axis=-1, keepdims=True)
    v = jnp.mean((x - m) ** 2, axis=-1, keepdims=True)
    return (x - m) / jnp.sqrt(v + 1e-5) * g + b


def _attn_head(emb3, attn, tgtf, W, b):
    embf = emb3.reshape(_BBLK * S, D)
    e = jnp.dot(embf, attn, preferred_element_type=_f32)
    e = jnp.where(e >= 0, e, 0.01 * e)
    e3 = e.reshape(_BBLK, S, H)
    m = jnp.max(e3, axis=1, keepdims=True)
    ex = jnp.exp(e3 - m)
    a = ex / jnp.sum(ex, axis=1, keepdims=True)
    parts = []
    for h in range(H):
        w = a[:, :, h]
        parts.append(jnp.sum(w[:, :, None] * emb3, axis=1) + tgtf)
    x = _elu(jnp.concatenate(parts, axis=1))
    return jnp.dot(x, W, preferred_element_type=_f32) + b


def _tail_body(eu_ref, ei_ref, tu_ref, ti_ref, au_ref, ai_ref,
               wu_ref, bu_ref, wi_ref, bi_ref,
               wx_ref, bx_ref, gx_ref, btx_ref,
               w1_ref, b1_ref, g1_ref, bt1_ref,
               w2_ref, b2_ref, o_ref):
    hu = _attn_head(eu_ref[...], au_ref[...], tu_ref[...],
                    wu_ref[...], bu_ref[...])
    hi = _attn_head(ei_ref[...], ai_ref[...], ti_ref[...],
                    wi_ref[...], bi_ref[...])
    x = hu * hi
    x = _elu(_ln_tc(jnp.dot(x, wx_ref[...], preferred_element_type=_f32)
                    + bx_ref[...], gx_ref[...], btx_ref[...]))
    y = _elu(_ln_tc(jnp.dot(x, w1_ref[...], preferred_element_type=_f32)
                    + b1_ref[...], g1_ref[...], bt1_ref[...]))
    z = jnp.dot(y, w2_ref[...], preferred_element_type=_f32) + b2_ref[...]
    o_ref[...] = 1.0 / (1.0 + jnp.exp(-z))


def _tail(embu3, embi3, tfu, tfi, attn_u, attn_i, Wu, bu, Wi, bi,
          Wx, bx, gx, btx, W1, b1, g1, bt1, W2p, b2p):
    full = lambda shape: pl.BlockSpec(shape, lambda i: tuple(0 for _ in shape))
    return pl.pallas_call(
        _tail_body,
        grid=(B // _BBLK,),
        in_specs=[
            pl.BlockSpec((_BBLK, S, D), lambda i: (i, 0, 0)),
            pl.BlockSpec((_BBLK, S, D), lambda i: (i, 0, 0)),
            pl.BlockSpec((_BBLK, D), lambda i: (i, 0)),
            pl.BlockSpec((_BBLK, D), lambda i: (i, 0)),
            full((D, H)), full((D, H)),
            full((H * D, D)), full((1, D)),
            full((H * D, D)), full((1, D)),
            full((D, D)), full((1, D)), full((1, D)), full((1, D)),
            full((D, D)), full((1, D)), full((1, D)), full((1, D)),
            full((D, D)), full((1, D)),
        ],
        out_specs=pl.BlockSpec((_BBLK, D), lambda i: (i, 0)),
        out_shape=jax.ShapeDtypeStruct((B, D), _f32),
    )(embu3, embi3, tfu, tfi, attn_u, attn_i, Wu, bu, Wi, bi,
      Wx, bx, gx, btx, W1, b1, g1, bt1, W2p, b2p)


def kernel(features_all, type_mask, edge_index, mp_idx_user, mp_idx_item,
           tgt_user, tgt_item, W_fc0, b_fc0, W_fc1, b_fc1, W_hg, attn_u,
           attn_i, W_user, b_user, W_item, b_item, W_xij, b_xij, g_xij,
           bt_xij, W_l1, b_l1, g_l1, bt_l1, W_l2, b_l2):
    i32 = jnp.int32

    xp = jnp.pad(features_all, ((0, NPAD - N), (0, 0)))
    tm = jnp.pad(type_mask.astype(_f32), (0, NPAD - N)).reshape(NPAD, 1)

    src = edge_index[0].astype(i32).reshape(NW, EPT)
    dst = edge_index[1].astype(i32).reshape(NW, EPT)
    packed = src | (dst << 14)
    ep = jnp.concatenate(
        [packed, jnp.full((NW, EPT_PAD - EPT), N << 14, i32)],
        axis=1).reshape(NW, KJ, 128)

    mpu = mp_idx_user.astype(i32).reshape(NW, _JB, 128, L).transpose(0, 3, 1, 2)
    mpi = mp_idx_item.astype(i32).reshape(NW, _JB, 128, L).transpose(0, 3, 1, 2)
    tgtu2 = tgt_user.astype(i32).reshape(NW, _TGT_PER_W)
    tgti2 = tgt_item.astype(i32).reshape(NW, _TGT_PER_W)

    feats = _project(xp, tm, W_fc0, b_fc0.reshape(1, D),
                     W_fc1, b_fc1.reshape(1, D))
    aggp, degp = _build_edge_agg()(feats, ep)
    feats2 = _hgcn(feats, aggp, degp.reshape(NC, NPAD, 1), W_hg)

    embu, embi, tfu, tfi = _build_mp_gather()(feats2, mpu, mpi, tgtu2, tgti2)
    embu3 = embu.reshape(B, S, D)
    embi3 = embi.reshape(B, S, D)

    W2p = jnp.pad(W_l2, ((0, 0), (0, D - 1)))
    b2p = jnp.pad(b_l2, (0, D - 1)).reshape(1, D)

    out2d = _tail(embu3, embi3, tfu, tfi, attn_u, attn_i,
                  W_user, b_user.reshape(1, D), W_item, b_item.reshape(1, D),
                  W_xij, b_xij.reshape(1, D), g_xij.reshape(1, D),
                  bt_xij.reshape(1, D),
                  W_l1, b_l1.reshape(1, D), g_l1.reshape(1, D),
                  bt_l1.reshape(1, D), W2p, b2p)
    return out2d[:, 0]
